# baseline (device time: 19597 ns/iter reference)
import jax
import jax.numpy as jnp
from jax import lax
from jax.experimental import pallas as pl
from jax.experimental.pallas import tpu as pltpu

NF = 16
FCH = 128
U = 64
NY = 12
NX = 10
NZ = 10


def kernel(x):
    m, n = x.shape
    n_half = n // 2
    q = m // 4

    def body(x_hbm, out_ref, x_vmem, send_buf, fetch_sems,
             ys, yr, xs, xr, zs, zr):
        my_x = lax.axis_index("x")
        my_y = lax.axis_index("y")
        my_z = lax.axis_index("z")
        pz = my_z % 2

        barrier = pltpu.get_barrier_semaphore()
        for dev in ((my_x, 1 - my_y, my_z),
                    (1 - my_x, my_y, my_z),
                    (my_x, my_y, my_z + 1 - 2 * pz)):
            pl.semaphore_signal(
                barrier, inc=1, device_id=dev,
                device_id_type=pl.DeviceIdType.MESH,
            )

        for xx in (0, 1):
            for yy in (0, 1):
                for pp in (0, 1):

                    @pl.when((my_x == xx) & (my_y == yy) & (pz == pp))
                    def _(xx=xx, yy=yy, pp=pp):
                        peer = (xx, 1 - yy, my_z)
                        xnbr = (1 - xx, yy, my_z)
                        prtn = (xx, yy, my_z + 1 - 2 * pp)

                        send_lo = (1 - yy) * n_half
                        keep_lo = yy * n_half
                        ob = (1 - yy) * m

                        own_lo = xx * 2 * q + pp * q
                        xn_lo = (1 - xx) * 2 * q + pp * q
                        zn_lo = xx * 2 * q + (1 - pp) * q
                        dg_lo = (1 - xx) * 2 * q + (1 - pp) * q

                        y_rows = [own_lo + u * U for u in range(8)] + \
                                 [dg_lo + v * U for v in range(4)]

                        need_first = sorted({r // FCH for r in y_rows})
                        order = need_first + [c for c in range(NF)
                                              if c not in need_first]
                        fetches = {}
                        for c in order:
                            cp = pltpu.make_async_copy(
                                x_hbm.at[pl.ds(c * FCH, FCH), :],
                                x_vmem.at[pl.ds(c * FCH, FCH), :],
                                fetch_sems.at[c],
                            )
                            cp.start()
                            fetches[c] = cp

                        waited = set()
                        for u, r in enumerate(y_rows):
                            c = r // FCH
                            if c not in waited:
                                fetches[c].wait()
                                waited.add(c)
                            send_buf[u * U:(u + 1) * U, :] = (
                                x_vmem[r:r + U, send_lo:send_lo + n_half]
                                .astype(jnp.bfloat16)
                            )

                        pl.semaphore_wait(barrier, 3)

                        y_rdmas = []
                        for u, r in enumerate(y_rows):
                            rdma = pltpu.make_async_remote_copy(
                                src_ref=send_buf.at[pl.ds(u * U, U), :],
                                dst_ref=out_ref.at[pl.ds(yy * m + r, U), :],
                                send_sem=ys.at[u], recv_sem=yr.at[u],
                                device_id=peer,
                                device_id_type=pl.DeviceIdType.MESH,
                            )
                            rdma.start()
                            y_rdmas.append(rdma)

                        def local_cast(c):
                            if c not in waited:
                                fetches[c].wait()
                                waited.add(c)
                            out_ref[yy * m + c * FCH:
                                    yy * m + (c + 1) * FCH, :] = (
                                x_vmem[c * FCH:(c + 1) * FCH,
                                       keep_lo:keep_lo + n_half]
                                .astype(jnp.bfloat16)
                            )

                        def fwd(row_lo, sems_s, sems_r, i, dev):
                            sl = pl.ds(ob + row_lo, U)
                            r = pltpu.make_async_remote_copy(
                                src_ref=out_ref.at[sl, :],
                                dst_ref=out_ref.at[sl, :],
                                send_sem=sems_s.at[i], recv_sem=sems_r.at[i],
                                device_id=dev,
                                device_id_type=pl.DeviceIdType.MESH,
                            )
                            r.start()
                            return r

                        x_rdmas = []
                        z_rdmas = []
                        for u in range(8):
                            y_rdmas[u].wait_recv()
                            x_rdmas.append(
                                fwd(own_lo + u * U, xs, xr, u, xnbr))
                            z_rdmas.append(
                                fwd(own_lo + u * U, zs, zr, u, prtn))
                            local_cast(order[2 * u])
                            local_cast(order[2 * u + 1])

                        for j in range(2):
                            z_rdmas[4 + j].wait_recv()
                            x_rdmas.append(
                                fwd(zn_lo + 256 + j * U, xs, xr, 8 + j,
                                    xnbr))
                        for j in range(2):
                            x_rdmas[6 + j].wait_recv()
                            z_rdmas.append(
                                fwd(xn_lo + 384 + j * U, zs, zr, 8 + j,
                                    prtn))

                        for r in y_rdmas:
                            r.wait_send()
                        for r in x_rdmas:
                            r.wait_send()
                        for r in z_rdmas:
                            r.wait_send()
                        for u in range(8, NY):
                            y_rdmas[u].wait_recv()
                        for i in (0, 1, 2, 3, 4, 5, 8, 9):
                            x_rdmas[i].wait_recv()
                        for i in (0, 1, 2, 3, 6, 7, 8, 9):
                            z_rdmas[i].wait_recv()

    return pl.pallas_call(
        body,
        out_shape=jax.ShapeDtypeStruct((2 * m, n_half), jnp.bfloat16),
        in_specs=[pl.BlockSpec(memory_space=pltpu.MemorySpace.HBM)],
        out_specs=pl.BlockSpec(memory_space=pltpu.VMEM),
        scratch_shapes=[
            pltpu.VMEM((m, n), jnp.float32),
            pltpu.VMEM((NY * U, n_half), jnp.bfloat16),
            pltpu.SemaphoreType.DMA((NF,)),
            pltpu.SemaphoreType.DMA((NY,)),
            pltpu.SemaphoreType.DMA((NY,)),
            pltpu.SemaphoreType.DMA((NX,)),
            pltpu.SemaphoreType.DMA((NX,)),
            pltpu.SemaphoreType.DMA((NZ,)),
            pltpu.SemaphoreType.DMA((NZ,)),
        ],
        compiler_params=pltpu.CompilerParams(collective_id=0),
    )(pltpu.with_memory_space_constraint(x, pltpu.MemorySpace.HBM))


# device time: 19444 ns/iter; 1.0079x vs baseline; 1.0079x over previous
import jax
import jax.numpy as jnp
from jax import lax
from jax.experimental import pallas as pl
from jax.experimental.pallas import tpu as pltpu

NF = 16
FCH = 128
U = 32
NU = 16
NY = 24
NX = 20
NZ = 20


def kernel(x):
    m, n = x.shape
    n_half = n // 2
    q = m // 4

    def body(x_hbm, out_ref, x_vmem, send_buf, fetch_sems,
             ys, yr, xs, xr, zs, zr):
        my_x = lax.axis_index("x")
        my_y = lax.axis_index("y")
        my_z = lax.axis_index("z")
        pz = my_z % 2

        barrier = pltpu.get_barrier_semaphore()
        for dev in ((my_x, 1 - my_y, my_z),
                    (1 - my_x, my_y, my_z),
                    (my_x, my_y, my_z + 1 - 2 * pz)):
            pl.semaphore_signal(
                barrier, inc=1, device_id=dev,
                device_id_type=pl.DeviceIdType.MESH,
            )

        for xx in (0, 1):
            for yy in (0, 1):
                for pp in (0, 1):

                    @pl.when((my_x == xx) & (my_y == yy) & (pz == pp))
                    def _(xx=xx, yy=yy, pp=pp):
                        peer = (xx, 1 - yy, my_z)
                        xnbr = (1 - xx, yy, my_z)
                        prtn = (xx, yy, my_z + 1 - 2 * pp)

                        send_lo = (1 - yy) * n_half
                        keep_lo = yy * n_half
                        ob = (1 - yy) * m

                        own_lo = xx * 2 * q + pp * q
                        xn_lo = (1 - xx) * 2 * q + pp * q
                        zn_lo = xx * 2 * q + (1 - pp) * q
                        dg_lo = (1 - xx) * 2 * q + (1 - pp) * q

                        y_rows = [own_lo + u * U for u in range(NU)] + \
                                 [dg_lo + v * U for v in range(8)]

                        need_first = sorted({r // FCH for r in y_rows})
                        order = need_first + [c for c in range(NF)
                                              if c not in need_first]
                        fetches = {}
                        for c in order:
                            cp = pltpu.make_async_copy(
                                x_hbm.at[pl.ds(c * FCH, FCH), :],
                                x_vmem.at[pl.ds(c * FCH, FCH), :],
                                fetch_sems.at[c],
                            )
                            cp.start()
                            fetches[c] = cp

                        waited = set()
                        for u, r in enumerate(y_rows):
                            c = r // FCH
                            if c not in waited:
                                fetches[c].wait()
                                waited.add(c)
                            send_buf[u * U:(u + 1) * U, :] = (
                                x_vmem[r:r + U, send_lo:send_lo + n_half]
                                .astype(jnp.bfloat16)
                            )

                        pl.semaphore_wait(barrier, 3)

                        y_rdmas = []
                        for u, r in enumerate(y_rows):
                            rdma = pltpu.make_async_remote_copy(
                                src_ref=send_buf.at[pl.ds(u * U, U), :],
                                dst_ref=out_ref.at[pl.ds(yy * m + r, U), :],
                                send_sem=ys.at[u], recv_sem=yr.at[u],
                                device_id=peer,
                                device_id_type=pl.DeviceIdType.MESH,
                            )
                            rdma.start()
                            y_rdmas.append(rdma)

                        def local_cast(c):
                            if c not in waited:
                                fetches[c].wait()
                                waited.add(c)
                            out_ref[yy * m + c * FCH:
                                    yy * m + (c + 1) * FCH, :] = (
                                x_vmem[c * FCH:(c + 1) * FCH,
                                       keep_lo:keep_lo + n_half]
                                .astype(jnp.bfloat16)
                            )

                        def fwd(row_lo, sems_s, sems_r, i, dev):
                            sl = pl.ds(ob + row_lo, U)
                            r = pltpu.make_async_remote_copy(
                                src_ref=out_ref.at[sl, :],
                                dst_ref=out_ref.at[sl, :],
                                send_sem=sems_s.at[i], recv_sem=sems_r.at[i],
                                device_id=dev,
                                device_id_type=pl.DeviceIdType.MESH,
                            )
                            r.start()
                            return r

                        x_rdmas = []
                        z_rdmas = []
                        for u in range(NU):
                            y_rdmas[u].wait_recv()
                            x_rdmas.append(
                                fwd(own_lo + u * U, xs, xr, u, xnbr))
                            z_rdmas.append(
                                fwd(own_lo + u * U, zs, zr, u, prtn))
                            local_cast(order[u])

                        for j in range(4):
                            z_rdmas[8 + j].wait_recv()
                            x_rdmas.append(
                                fwd(zn_lo + 256 + j * U, xs, xr, NU + j,
                                    xnbr))
                        for j in range(4):
                            x_rdmas[12 + j].wait_recv()
                            z_rdmas.append(
                                fwd(xn_lo + 384 + j * U, zs, zr, NU + j,
                                    prtn))

                        for r in y_rdmas:
                            r.wait_send()
                        for r in x_rdmas:
                            r.wait_send()
                        for r in z_rdmas:
                            r.wait_send()
                        for u in range(NU, NY):
                            y_rdmas[u].wait_recv()
                        for i in list(range(12)) + list(range(NU, NX)):
                            x_rdmas[i].wait_recv()
                        for i in (list(range(8)) + list(range(12, 16))
                                  + list(range(NU, NZ))):
                            z_rdmas[i].wait_recv()

    return pl.pallas_call(
        body,
        out_shape=jax.ShapeDtypeStruct((2 * m, n_half), jnp.bfloat16),
        in_specs=[pl.BlockSpec(memory_space=pltpu.MemorySpace.HBM)],
        out_specs=pl.BlockSpec(memory_space=pltpu.VMEM),
        scratch_shapes=[
            pltpu.VMEM((m, n), jnp.float32),
            pltpu.VMEM((NY * U, n_half), jnp.bfloat16),
            pltpu.SemaphoreType.DMA((NF,)),
            pltpu.SemaphoreType.DMA((NY,)),
            pltpu.SemaphoreType.DMA((NY,)),
            pltpu.SemaphoreType.DMA((NX,)),
            pltpu.SemaphoreType.DMA((NX,)),
            pltpu.SemaphoreType.DMA((NZ,)),
            pltpu.SemaphoreType.DMA((NZ,)),
        ],
        compiler_params=pltpu.CompilerParams(collective_id=0),
    )(pltpu.with_memory_space_constraint(x, pltpu.MemorySpace.HBM))
